# baseline (speedup 1.0000x reference)
import jax
import jax.numpy as jnp
from jax.experimental import pallas as pl
from jax.experimental.pallas import tpu as pltpu
from jax.experimental.pallas import tpu_sc as plsc

B, T, H = 2, 2048, 1024
L = H // 4
I = int(H * 2.0)
NS = 2
NR = 8 - NS
NE = NR + NS
TOPK = 2
BASE = 10000.0
SCALE = 1.0
EPS = 1e-5
N = B * T

BQ = 256
BR = 256
RL = 128
BKR = 256
NBP = (N * TOPK) // BKR + NR
PN = NBP * BKR
GW = 128

_f32 = jnp.float32
_bf16 = jnp.bfloat16


def _ln(x, w):
    mu = jnp.mean(x, axis=1, keepdims=True)
    xc = x - mu
    var = jnp.mean(xc * xc, axis=1, keepdims=True)
    return xc * jax.lax.rsqrt(var + EPS) * w


def _prep_kernel(x_ref, ln1w_ref, wqd_ref, wrq_ref, wrk_ref, wkvd_ref,
                 wvu_ref, cos_ref, sin_ref, q_ref, k_ref, v_ref):
    x = x_ref[...]
    xb = _ln(x, ln1w_ref[...]).astype(_bf16)
    ql = jnp.dot(xb, wqd_ref[...], preferred_element_type=_f32).astype(_bf16)
    qr = jnp.dot(ql, wrq_ref[...], preferred_element_type=_f32)
    kr = jnp.dot(xb, wrk_ref[...], preferred_element_type=_f32)
    kv = jnp.dot(xb, wkvd_ref[...], preferred_element_type=_f32).astype(_bf16)
    v = jnp.dot(kv, wvu_ref[...], preferred_element_type=_f32)
    cos = cos_ref[...]
    sin = sin_ref[...]

    def rope(t):
        t1 = t[:, :H // 2]
        t2 = t[:, H // 2:]
        rot = jnp.concatenate([-t2, t1], axis=1)
        return t * cos + rot * sin

    q_ref[...] = rope(qr).astype(_bf16)
    k_ref[...] = rope(kr).astype(_bf16)
    v_ref[...] = v.astype(_bf16)


def _attn_kernel(q_ref, k_ref, v_ref, o_ref):
    qi = pl.program_id(1)
    q = q_ref[0]
    k = k_ref[0]
    s = jax.lax.dot_general(q, k, (((1,), (1,)), ((), ())),
                            preferred_element_type=_f32) * (1.0 / 32.0)
    row = qi * BQ + jax.lax.broadcasted_iota(jnp.int32, (BQ, T), 0)
    col = jax.lax.broadcasted_iota(jnp.int32, (BQ, T), 1)
    s = jnp.where(row >= col, s, -1e30)
    m = jnp.max(s, axis=1, keepdims=True)
    p = jnp.exp(s - m)
    p = p / jnp.sum(p, axis=1, keepdims=True)
    o_ref[0] = jnp.dot(p.astype(_bf16), v_ref[0],
                       preferred_element_type=_f32).astype(_bf16)


def _post_kernel(y_ref, x_ref, wo_ref, ln2w_ref, wrt_ref, bias_ref,
                 h_ref, xn2_ref, logits_ref, lt_ref):
    h = x_ref[...] + jnp.dot(y_ref[...], wo_ref[...],
                             preferred_element_type=_f32)
    h_ref[...] = h
    xn2 = _ln(h, ln2w_ref[...])
    xn2_ref[...] = xn2
    logits = jnp.dot(xn2, wrt_ref[...],
                     preferred_element_type=_f32) + bias_ref[...]
    logits_ref[...] = logits
    lt_ref[...] = jnp.transpose(logits[:, :8])


def _shared_kernel(xn2_ref, h_ref, gate_ref, up_ref, down_ref, out_ref):
    e = pl.program_id(1)
    xb = xn2_ref[...].astype(_bf16)
    a = jnp.dot(xb, gate_ref[0], preferred_element_type=_f32)
    b = jnp.dot(xb, up_ref[0], preferred_element_type=_f32)
    h1 = (a * jax.nn.sigmoid(a) * b).astype(_bf16)
    contrib = jnp.dot(h1, down_ref[0],
                      preferred_element_type=_f32) * (1.0 / NS)

    @pl.when(e == 0)
    def _init():
        out_ref[...] = h_ref[...] + contrib

    @pl.when(e != 0)
    def _acc():
        out_ref[...] += contrib


def _cumsum_lanes(x):
    n = x.shape[-1]
    sh = 1
    while sh < n:
        shifted = jnp.concatenate(
            [jnp.zeros((x.shape[0], sh), x.dtype), x[:, :n - sh]], axis=1)
        x = x + shifted
        sh *= 2
    return x


def _routing_kernel(lt_ref, pos0_ref, pos1_ref, w0_ref, w1_ref, beid_ref):
    P = jax.nn.sigmoid(lt_ref[...])
    eio = jax.lax.broadcasted_iota(jnp.int32, (8, N), 0)
    valid = eio < NR
    Pm = jnp.where(valid, P, -1.0)
    m1 = jnp.max(Pm, axis=0, keepdims=True)
    e0 = jnp.min(jnp.where(Pm == m1, eio, NR), axis=0, keepdims=True)
    mask0 = eio == e0
    Pm2 = jnp.where(mask0, -1.0, Pm)
    m2 = jnp.max(Pm2, axis=0, keepdims=True)
    e1 = jnp.min(jnp.where(Pm2 == m2, eio, NR), axis=0, keepdims=True)
    mask1 = eio == e1

    ind0 = mask0.astype(jnp.int32)
    ind1 = mask1.astype(jnp.int32)
    c0 = _cumsum_lanes(ind0) - ind0
    c1 = _cumsum_lanes(ind1) - ind1
    tot0 = jnp.sum(ind0, axis=1, keepdims=True)
    tot1 = jnp.sum(ind1, axis=1, keepdims=True)
    cnt = tot0 + tot1
    pcnt = ((cnt + BKR - 1) // BKR) * BKR

    offs = [jnp.zeros((1, 1), jnp.int32)]
    for e in range(1, NR):
        offs.append(offs[-1] + pcnt[e - 1:e, :])
    offs += [offs[-1] + pcnt[NR - 1:NR, :]] * (8 - NR)
    poff = jnp.concatenate(offs, axis=0)

    rank0 = jnp.sum(jnp.where(mask0, c0, 0), axis=0, keepdims=True)
    rank1 = jnp.sum(jnp.where(mask1, tot0 + c1, 0), axis=0, keepdims=True)
    base0 = jnp.sum(jnp.where(mask0, poff, 0), axis=0, keepdims=True)
    base1 = jnp.sum(jnp.where(mask1, poff, 0), axis=0, keepdims=True)
    pos0_ref[...] = base0 + rank0
    pos1_ref[...] = base1 + rank1
    w0_ref[...] = m1
    w1_ref[...] = m2

    bio = jax.lax.broadcasted_iota(jnp.int32, (1, NBP), 1)
    bstart = bio * BKR
    eid = jnp.full((1, NBP), -1, jnp.int32)
    for e in range(NR):
        pe = poff[e:e + 1, :]
        in_e = (bstart >= pe) & (bstart < pe + pcnt[e:e + 1, :])
        eid = jnp.where(in_e, e, eid)
    beid_ref[...] = eid


def _grouped_ffn_kernel(s_ref, xs_ref, gate_ref, up_ref, down_ref, ys_ref):
    eid = s_ref[pl.program_id(0)]

    @pl.when(eid >= 0)
    def _compute():
        xb = xs_ref[...].astype(_bf16)
        a = jnp.dot(xb, gate_ref[0], preferred_element_type=_f32)
        b = jnp.dot(xb, up_ref[0], preferred_element_type=_f32)
        h1 = (a * jax.nn.sigmoid(a) * b).astype(_bf16)
        ys_ref[...] = jnp.dot(h1, down_ref[0], preferred_element_type=_f32)

    @pl.when(eid < 0)
    def _skip():
        ys_ref[...] = jnp.zeros((BKR, H), _f32)


def _combine_kernel(base_ref, g0_ref, g1_ref, w0_ref, w1_ref, out_ref):
    w0 = jnp.transpose(w0_ref[...])
    w1 = jnp.transpose(w1_ref[...])
    out_ref[...] = (base_ref[...]
                    + w0 * g0_ref[...].astype(_f32)
                    + w1 * g1_ref[...].astype(_f32))


HQ = H // 4
N2 = 4 * N
PN2 = 4 * PN


def _sc_mesh():
    return plsc.VectorSubcoreMesh(core_axis_name="core",
                                  subcore_axis_name="subcore")


def _sc_scatter_rows(x2, i0x, i1x):
    @pl.kernel(out_type=jax.ShapeDtypeStruct((PN2, HQ), _f32),
               mesh=_sc_mesh())
    def _k(x_hbm, p0_hbm, p1_hbm, o_hbm):
        def body(x_vmem, i0_vmem, i1_vmem):
            pltpu.sync_copy(x_vmem, o_hbm.at[i0_vmem.at[0]])
            pltpu.sync_copy(x_vmem, o_hbm.at[i1_vmem.at[0]])

        pltpu.emit_pipeline(
            body,
            grid=(N2 // GW,),
            in_specs=[
                pl.BlockSpec((GW, HQ), lambda i: (i, 0)),
                pl.BlockSpec((1, GW), lambda i: (0, i)),
                pl.BlockSpec((1, GW), lambda i: (0, i)),
            ],
            out_specs=[],
            core_axis_name="subcore",
            dimension_semantics=(pltpu.PARALLEL,),
        )(x_hbm, p0_hbm, p1_hbm)

    return _k(x2, i0x, i1x)


def _sc_gather_rows(y2, ix):
    @pl.kernel(out_type=jax.ShapeDtypeStruct((N2, HQ), _f32),
               mesh=_sc_mesh())
    def _k(y_hbm, p_hbm, o_hbm):
        def body(i_vmem, o_vmem):
            pltpu.sync_copy(y_hbm.at[i_vmem.at[0]], o_vmem)

        pltpu.emit_pipeline(
            body,
            grid=(N2 // GW,),
            in_specs=[pl.BlockSpec((1, GW), lambda i: (0, i))],
            out_specs=[pl.BlockSpec((GW, HQ), lambda i: (i, 0))],
            core_axis_name="subcore",
            dimension_semantics=(pltpu.PARALLEL,),
        )(p_hbm, o_hbm)

    return _k(y2, ix)


def _qrow_indices(pos):
    p = pos.reshape(N, 1)
    return (4 * p + jnp.arange(4, dtype=jnp.int32)[None, :]).reshape(1, N2)


def kernel(x, ln1_w, ln2_w, W_kvd, W_qd, W_ku, W_qu, W_vu, W_rk, W_rq, W_o,
           sh_gate, sh_up, sh_down, rt_gate, rt_up, rt_down, W_router,
           routing_bias):
    del W_ku, W_qu
    xf = x.reshape(N, H)

    wqd_t = W_qd.T.astype(_bf16)
    wrq_t = W_rq.T.astype(_bf16)
    wrk_t = W_rk.T.astype(_bf16)
    wkvd_t = W_kvd.T.astype(_bf16)
    wvu_t = W_vu.T.astype(_bf16)
    wo_t = W_o.T.astype(_bf16)
    ln1w2 = ln1_w.reshape(1, H)
    ln2w2 = ln2_w.reshape(1, H)

    inv_freq = 1.0 / BASE ** (jnp.arange(0, H, 2, dtype=_f32) / H)
    tt = jnp.arange(T, dtype=_f32)
    freqs = tt[:, None] * inv_freq[None, :]
    emb = jnp.concatenate([freqs, freqs], axis=-1)
    cos = jnp.cos(emb) * SCALE
    sin = jnp.sin(emb) * SCALE

    wrt_t = jnp.zeros((H, RL), _f32).at[:, :NR].set(W_router.T)
    bias_p = jnp.full((1, RL), -1e30, _f32).at[0, :NR].set(routing_bias)

    all_gate_t = jnp.concatenate([rt_gate, sh_gate]).transpose(0, 2, 1).astype(_bf16)
    all_up_t = jnp.concatenate([rt_up, sh_up]).transpose(0, 2, 1).astype(_bf16)
    all_down_t = jnp.concatenate([rt_down, sh_down]).transpose(0, 2, 1).astype(_bf16)

    nrow = N // BQ
    row_spec = pl.BlockSpec((BQ, H), lambda i: (i, 0))
    full = lambda shape: pl.BlockSpec(shape, lambda i: tuple(0 for _ in shape))
    cs_spec = pl.BlockSpec((BQ, H), lambda i: (i % (T // BQ), 0))
    q, k, v = pl.pallas_call(
        _prep_kernel,
        grid=(nrow,),
        in_specs=[row_spec, full((1, H)), full((H, L)), full((L, H)),
                  full((H, H)), full((H, L)), full((L, H)), cs_spec, cs_spec],
        out_specs=[row_spec, row_spec, row_spec],
        out_shape=[jax.ShapeDtypeStruct((N, H), _bf16)] * 3,
        compiler_params=pltpu.CompilerParams(
            dimension_semantics=("parallel",)),
    )(xf, ln1w2, wqd_t, wrq_t, wrk_t, wkvd_t, wvu_t, cos, sin)

    q3 = q.reshape(B, T, H)
    k3 = k.reshape(B, T, H)
    v3 = v.reshape(B, T, H)
    qb_spec = pl.BlockSpec((1, BQ, H), lambda b, i: (b, i, 0))
    kv_spec = pl.BlockSpec((1, T, H), lambda b, i: (b, 0, 0))
    y = pl.pallas_call(
        _attn_kernel,
        grid=(B, T // BQ),
        in_specs=[qb_spec, kv_spec, kv_spec],
        out_specs=qb_spec,
        out_shape=jax.ShapeDtypeStruct((B, T, H), _bf16),
        compiler_params=pltpu.CompilerParams(
            dimension_semantics=("parallel", "parallel")),
    )(q3, k3, v3)

    h, xn2, logits, logitsT = pl.pallas_call(
        _post_kernel,
        grid=(nrow,),
        in_specs=[row_spec, row_spec, full((H, H)), full((1, H)),
                  full((H, RL)), full((1, RL))],
        out_specs=[row_spec, row_spec,
                   pl.BlockSpec((BQ, RL), lambda i: (i, 0)),
                   pl.BlockSpec((8, BQ), lambda i: (0, i))],
        out_shape=[jax.ShapeDtypeStruct((N, H), _f32),
                   jax.ShapeDtypeStruct((N, H), _f32),
                   jax.ShapeDtypeStruct((N, RL), _f32),
                   jax.ShapeDtypeStruct((8, N), _f32)],
        compiler_params=pltpu.CompilerParams(
            dimension_semantics=("parallel",)),
    )(y.reshape(N, H), xf, wo_t, ln2w2, wrt_t, bias_p)

    pos0, pos1, w0, w1, beid = pl.pallas_call(
        _routing_kernel,
        grid=(1,),
        in_specs=[pl.BlockSpec((8, N), lambda i: (0, 0))],
        out_specs=[pl.BlockSpec((1, N), lambda i: (0, 0))] * 4
        + [pl.BlockSpec((1, NBP), lambda i: (0, 0))],
        out_shape=[jax.ShapeDtypeStruct((1, N), jnp.int32)] * 2
        + [jax.ShapeDtypeStruct((1, N), _f32)] * 2
        + [jax.ShapeDtypeStruct((1, NBP), jnp.int32)],
    )(logitsT)
    beid_1d = beid.reshape(NBP)

    nrow_m = N // BR
    rspec = pl.BlockSpec((BR, H), lambda r, e: (r, 0))
    base = pl.pallas_call(
        _shared_kernel,
        grid=(nrow_m, NS),
        in_specs=[rspec, rspec,
                  pl.BlockSpec((1, H, I), lambda r, e: (e, 0, 0)),
                  pl.BlockSpec((1, H, I), lambda r, e: (e, 0, 0)),
                  pl.BlockSpec((1, I, H), lambda r, e: (e, 0, 0))],
        out_specs=rspec,
        out_shape=jax.ShapeDtypeStruct((N, H), _f32),
        compiler_params=pltpu.CompilerParams(
            dimension_semantics=("parallel", "arbitrary")),
    )(xn2, h, all_gate_t[NR:], all_up_t[NR:], all_down_t[NR:])

    i0x = _qrow_indices(pos0)
    i1x = _qrow_indices(pos1)
    xs2 = _sc_scatter_rows(xn2.reshape(N2, HQ), i0x, i1x)
    xs = xs2.reshape(PN, H)

    ys = pl.pallas_call(
        _grouped_ffn_kernel,
        grid_spec=pltpu.PrefetchScalarGridSpec(
            num_scalar_prefetch=1,
            grid=(NBP,),
            in_specs=[
                pl.BlockSpec((BKR, H), lambda i, s: (i, 0)),
                pl.BlockSpec((1, H, I),
                             lambda i, s: (jnp.maximum(s[i], 0), 0, 0)),
                pl.BlockSpec((1, H, I),
                             lambda i, s: (jnp.maximum(s[i], 0), 0, 0)),
                pl.BlockSpec((1, I, H),
                             lambda i, s: (jnp.maximum(s[i], 0), 0, 0)),
            ],
            out_specs=pl.BlockSpec((BKR, H), lambda i, s: (i, 0)),
        ),
        out_shape=jax.ShapeDtypeStruct((PN, H), _f32),
    )(beid_1d, xs, all_gate_t[:NR], all_up_t[:NR], all_down_t[:NR])

    ys2 = ys.reshape(PN2, HQ)
    g0 = _sc_gather_rows(ys2, i0x).reshape(N, H)
    g1 = _sc_gather_rows(ys2, i1x).reshape(N, H)
    wspec = pl.BlockSpec((1, BQ), lambda i: (0, i))
    out = pl.pallas_call(
        _combine_kernel,
        grid=(nrow,),
        in_specs=[row_spec, row_spec, row_spec, wspec, wspec],
        out_specs=row_spec,
        out_shape=jax.ShapeDtypeStruct((N, H), _f32),
        compiler_params=pltpu.CompilerParams(
            dimension_semantics=("parallel",)),
    )(base, g0, g1, w0, w1)

    return out.reshape(B, T, H)


# speedup vs baseline: 1.0334x; 1.0334x over previous
import jax
import jax.numpy as jnp
from jax.experimental import pallas as pl
from jax.experimental.pallas import tpu as pltpu
from jax.experimental.pallas import tpu_sc as plsc

B, T, H = 2, 2048, 1024
L = H // 4
I = int(H * 2.0)
NS = 2
NR = 8 - NS
NE = NR + NS
TOPK = 2
BASE = 10000.0
SCALE = 1.0
EPS = 1e-5
N = B * T

BQ = 256
BR = 256
RL = 128
BKR = 256
NBP = (N * TOPK) // BKR + NR
PN = NBP * BKR
GW = 128

_f32 = jnp.float32
_bf16 = jnp.bfloat16


def _ln(x, w):
    mu = jnp.mean(x, axis=1, keepdims=True)
    xc = x - mu
    var = jnp.mean(xc * xc, axis=1, keepdims=True)
    return xc * jax.lax.rsqrt(var + EPS) * w


def _prep_kernel(x_ref, ln1w_ref, wqd_ref, wrq_ref, wrk_ref, wkvd_ref,
                 wvu_ref, cos_ref, sin_ref, q_ref, k_ref, v_ref):
    x = x_ref[...]
    xb = _ln(x, ln1w_ref[...]).astype(_bf16)
    ql = jnp.dot(xb, wqd_ref[...], preferred_element_type=_f32).astype(_bf16)
    qr = jnp.dot(ql, wrq_ref[...], preferred_element_type=_f32)
    kr = jnp.dot(xb, wrk_ref[...], preferred_element_type=_f32)
    kv = jnp.dot(xb, wkvd_ref[...], preferred_element_type=_f32).astype(_bf16)
    v = jnp.dot(kv, wvu_ref[...], preferred_element_type=_f32)
    cos = cos_ref[...]
    sin = sin_ref[...]

    def rope(t):
        t1 = t[:, :H // 2]
        t2 = t[:, H // 2:]
        rot = jnp.concatenate([-t2, t1], axis=1)
        return t * cos + rot * sin

    q_ref[...] = rope(qr).astype(_bf16)
    k_ref[...] = rope(kr).astype(_bf16)
    v_ref[...] = v.astype(_bf16)


def _attn_kernel(q_ref, k_ref, v_ref, o_ref):
    qi = pl.program_id(1)
    q = q_ref[0]
    k = k_ref[0]
    s = jax.lax.dot_general(q, k, (((1,), (1,)), ((), ())),
                            preferred_element_type=_f32) * (1.0 / 32.0)
    row = qi * BQ + jax.lax.broadcasted_iota(jnp.int32, (BQ, T), 0)
    col = jax.lax.broadcasted_iota(jnp.int32, (BQ, T), 1)
    s = jnp.where(row >= col, s, -1e30)
    m = jnp.max(s, axis=1, keepdims=True)
    p = jnp.exp(s - m)
    p = p / jnp.sum(p, axis=1, keepdims=True)
    o_ref[0] = jnp.dot(p.astype(_bf16), v_ref[0],
                       preferred_element_type=_f32).astype(_bf16)


def _post_kernel(y_ref, x_ref, wo_ref, ln2w_ref, wrt_ref, bias_ref,
                 h_ref, xn2_ref, logits_ref, lt_ref):
    h = x_ref[...] + jnp.dot(y_ref[...], wo_ref[...],
                             preferred_element_type=_f32)
    h_ref[...] = h
    xn2 = _ln(h, ln2w_ref[...])
    xn2_ref[...] = xn2
    logits = jnp.dot(xn2, wrt_ref[...],
                     preferred_element_type=_f32) + bias_ref[...]
    logits_ref[...] = logits
    lt_ref[...] = jnp.transpose(logits[:, :8])


def _shared_kernel(xn2_ref, h_ref, gate_ref, up_ref, down_ref, out_ref):
    e = pl.program_id(0)
    r = pl.program_id(1)
    xb = xn2_ref[...].astype(_bf16)
    a = jnp.dot(xb, gate_ref[0], preferred_element_type=_f32)
    b = jnp.dot(xb, up_ref[0], preferred_element_type=_f32)
    h1 = (a * jax.nn.sigmoid(a) * b).astype(_bf16)
    contrib = jnp.dot(h1, down_ref[0],
                      preferred_element_type=_f32) * (1.0 / NS)
    rows = pl.ds(r * BR, BR)

    @pl.when(e == 0)
    def _init():
        out_ref[rows, :] = h_ref[...] + contrib

    @pl.when(e != 0)
    def _acc():
        out_ref[rows, :] += contrib


def _cumsum_lanes(x):
    n = x.shape[-1]
    sh = 1
    while sh < n:
        shifted = jnp.concatenate(
            [jnp.zeros((x.shape[0], sh), x.dtype), x[:, :n - sh]], axis=1)
        x = x + shifted
        sh *= 2
    return x


def _routing_kernel(lt_ref, pos0_ref, pos1_ref, w0_ref, w1_ref, beid_ref):
    P = jax.nn.sigmoid(lt_ref[...])
    eio = jax.lax.broadcasted_iota(jnp.int32, (8, N), 0)
    valid = eio < NR
    Pm = jnp.where(valid, P, -1.0)
    m1 = jnp.max(Pm, axis=0, keepdims=True)
    e0 = jnp.min(jnp.where(Pm == m1, eio, NR), axis=0, keepdims=True)
    mask0 = eio == e0
    Pm2 = jnp.where(mask0, -1.0, Pm)
    m2 = jnp.max(Pm2, axis=0, keepdims=True)
    e1 = jnp.min(jnp.where(Pm2 == m2, eio, NR), axis=0, keepdims=True)
    mask1 = eio == e1

    ind0 = mask0.astype(jnp.int32)
    ind1 = mask1.astype(jnp.int32)
    c0 = _cumsum_lanes(ind0) - ind0
    c1 = _cumsum_lanes(ind1) - ind1
    tot0 = jnp.sum(ind0, axis=1, keepdims=True)
    tot1 = jnp.sum(ind1, axis=1, keepdims=True)
    cnt = tot0 + tot1
    pcnt = ((cnt + BKR - 1) // BKR) * BKR

    offs = [jnp.zeros((1, 1), jnp.int32)]
    for e in range(1, NR):
        offs.append(offs[-1] + pcnt[e - 1:e, :])
    offs += [offs[-1] + pcnt[NR - 1:NR, :]] * (8 - NR)
    poff = jnp.concatenate(offs, axis=0)

    rank0 = jnp.sum(jnp.where(mask0, c0, 0), axis=0, keepdims=True)
    rank1 = jnp.sum(jnp.where(mask1, tot0 + c1, 0), axis=0, keepdims=True)
    base0 = jnp.sum(jnp.where(mask0, poff, 0), axis=0, keepdims=True)
    base1 = jnp.sum(jnp.where(mask1, poff, 0), axis=0, keepdims=True)
    pos0_ref[...] = base0 + rank0
    pos1_ref[...] = base1 + rank1
    w0_ref[...] = m1
    w1_ref[...] = m2

    bio = jax.lax.broadcasted_iota(jnp.int32, (1, NBP), 1)
    bstart = bio * BKR
    eid = jnp.full((1, NBP), -1, jnp.int32)
    for e in range(NR):
        pe = poff[e:e + 1, :]
        in_e = (bstart >= pe) & (bstart < pe + pcnt[e:e + 1, :])
        eid = jnp.where(in_e, e, eid)
    beid_ref[...] = eid


def _grouped_ffn_kernel(s_ref, xs_ref, gate_ref, up_ref, down_ref, ys_ref):
    eid = s_ref[pl.program_id(0)]

    @pl.when(eid >= 0)
    def _compute():
        xb = xs_ref[...].astype(_bf16)
        a = jnp.dot(xb, gate_ref[0], preferred_element_type=_f32)
        b = jnp.dot(xb, up_ref[0], preferred_element_type=_f32)
        h1 = (a * jax.nn.sigmoid(a) * b).astype(_bf16)
        ys_ref[...] = jnp.dot(h1, down_ref[0], preferred_element_type=_f32)

    @pl.when(eid < 0)
    def _skip():
        ys_ref[...] = jnp.zeros((BKR, H), _f32)


def _combine_kernel(base_ref, g0_ref, g1_ref, w0_ref, w1_ref, out_ref):
    w0 = jnp.transpose(w0_ref[...])
    w1 = jnp.transpose(w1_ref[...])
    out_ref[...] = (base_ref[...]
                    + w0 * g0_ref[...].astype(_f32)
                    + w1 * g1_ref[...].astype(_f32))


HQ = H // 4
N2 = 4 * N
PN2 = 4 * PN


def _sc_mesh():
    return plsc.VectorSubcoreMesh(core_axis_name="core",
                                  subcore_axis_name="subcore")


def _sc_scatter_rows(x2, i0x, i1x):
    @pl.kernel(out_type=jax.ShapeDtypeStruct((PN2, HQ), _f32),
               mesh=_sc_mesh())
    def _k(x_hbm, p0_hbm, p1_hbm, o_hbm):
        def body(x_vmem, i0_vmem, i1_vmem):
            pltpu.sync_copy(x_vmem, o_hbm.at[i0_vmem.at[0]])
            pltpu.sync_copy(x_vmem, o_hbm.at[i1_vmem.at[0]])

        pltpu.emit_pipeline(
            body,
            grid=(N2 // GW,),
            in_specs=[
                pl.BlockSpec((GW, HQ), lambda i: (i, 0)),
                pl.BlockSpec((1, GW), lambda i: (0, i)),
                pl.BlockSpec((1, GW), lambda i: (0, i)),
            ],
            out_specs=[],
            core_axis_name="subcore",
            dimension_semantics=(pltpu.PARALLEL,),
        )(x_hbm, p0_hbm, p1_hbm)

    return _k(x2, i0x, i1x)


def _sc_gather_rows(y2, ix):
    @pl.kernel(out_type=jax.ShapeDtypeStruct((N2, HQ), _f32),
               mesh=_sc_mesh())
    def _k(y_hbm, p_hbm, o_hbm):
        def body(i_vmem, o_vmem):
            pltpu.sync_copy(y_hbm.at[i_vmem.at[0]], o_vmem)

        pltpu.emit_pipeline(
            body,
            grid=(N2 // GW,),
            in_specs=[pl.BlockSpec((1, GW), lambda i: (0, i))],
            out_specs=[pl.BlockSpec((GW, HQ), lambda i: (i, 0))],
            core_axis_name="subcore",
            dimension_semantics=(pltpu.PARALLEL,),
        )(p_hbm, o_hbm)

    return _k(y2, ix)


def _qrow_indices(pos):
    p = pos.reshape(N, 1)
    return (4 * p + jnp.arange(4, dtype=jnp.int32)[None, :]).reshape(1, N2)


def kernel(x, ln1_w, ln2_w, W_kvd, W_qd, W_ku, W_qu, W_vu, W_rk, W_rq, W_o,
           sh_gate, sh_up, sh_down, rt_gate, rt_up, rt_down, W_router,
           routing_bias):
    del W_ku, W_qu
    xf = x.reshape(N, H)

    wqd_t = W_qd.T.astype(_bf16)
    wrq_t = W_rq.T.astype(_bf16)
    wrk_t = W_rk.T.astype(_bf16)
    wkvd_t = W_kvd.T.astype(_bf16)
    wvu_t = W_vu.T.astype(_bf16)
    wo_t = W_o.T.astype(_bf16)
    ln1w2 = ln1_w.reshape(1, H)
    ln2w2 = ln2_w.reshape(1, H)

    inv_freq = 1.0 / BASE ** (jnp.arange(0, H, 2, dtype=_f32) / H)
    tt = jnp.arange(T, dtype=_f32)
    freqs = tt[:, None] * inv_freq[None, :]
    emb = jnp.concatenate([freqs, freqs], axis=-1)
    cos = jnp.cos(emb) * SCALE
    sin = jnp.sin(emb) * SCALE

    wrt_t = jnp.zeros((H, RL), _f32).at[:, :NR].set(W_router.T)
    bias_p = jnp.full((1, RL), -1e30, _f32).at[0, :NR].set(routing_bias)

    all_gate_t = jnp.concatenate([rt_gate, sh_gate]).transpose(0, 2, 1).astype(_bf16)
    all_up_t = jnp.concatenate([rt_up, sh_up]).transpose(0, 2, 1).astype(_bf16)
    all_down_t = jnp.concatenate([rt_down, sh_down]).transpose(0, 2, 1).astype(_bf16)

    nrow = N // BQ
    row_spec = pl.BlockSpec((BQ, H), lambda i: (i, 0))
    full = lambda shape: pl.BlockSpec(shape, lambda i: tuple(0 for _ in shape))
    cs_spec = pl.BlockSpec((BQ, H), lambda i: (i % (T // BQ), 0))
    q, k, v = pl.pallas_call(
        _prep_kernel,
        grid=(nrow,),
        in_specs=[row_spec, full((1, H)), full((H, L)), full((L, H)),
                  full((H, H)), full((H, L)), full((L, H)), cs_spec, cs_spec],
        out_specs=[row_spec, row_spec, row_spec],
        out_shape=[jax.ShapeDtypeStruct((N, H), _bf16)] * 3,
        compiler_params=pltpu.CompilerParams(
            dimension_semantics=("parallel",)),
    )(xf, ln1w2, wqd_t, wrq_t, wrk_t, wkvd_t, wvu_t, cos, sin)

    q3 = q.reshape(B, T, H)
    k3 = k.reshape(B, T, H)
    v3 = v.reshape(B, T, H)
    qb_spec = pl.BlockSpec((1, BQ, H), lambda b, i: (b, i, 0))
    kv_spec = pl.BlockSpec((1, T, H), lambda b, i: (b, 0, 0))
    y = pl.pallas_call(
        _attn_kernel,
        grid=(B, T // BQ),
        in_specs=[qb_spec, kv_spec, kv_spec],
        out_specs=qb_spec,
        out_shape=jax.ShapeDtypeStruct((B, T, H), _bf16),
        compiler_params=pltpu.CompilerParams(
            dimension_semantics=("parallel", "parallel")),
    )(q3, k3, v3)

    h, xn2, logits, logitsT = pl.pallas_call(
        _post_kernel,
        grid=(nrow,),
        in_specs=[row_spec, row_spec, full((H, H)), full((1, H)),
                  full((H, RL)), full((1, RL))],
        out_specs=[row_spec, row_spec,
                   pl.BlockSpec((BQ, RL), lambda i: (i, 0)),
                   pl.BlockSpec((8, BQ), lambda i: (0, i))],
        out_shape=[jax.ShapeDtypeStruct((N, H), _f32),
                   jax.ShapeDtypeStruct((N, H), _f32),
                   jax.ShapeDtypeStruct((N, RL), _f32),
                   jax.ShapeDtypeStruct((8, N), _f32)],
        compiler_params=pltpu.CompilerParams(
            dimension_semantics=("parallel",)),
    )(y.reshape(N, H), xf, wo_t, ln2w2, wrt_t, bias_p)

    pos0, pos1, w0, w1, beid = pl.pallas_call(
        _routing_kernel,
        grid=(1,),
        in_specs=[pl.BlockSpec((8, N), lambda i: (0, 0))],
        out_specs=[pl.BlockSpec((1, N), lambda i: (0, 0))] * 4
        + [pl.BlockSpec((1, NBP), lambda i: (0, 0))],
        out_shape=[jax.ShapeDtypeStruct((1, N), jnp.int32)] * 2
        + [jax.ShapeDtypeStruct((1, N), _f32)] * 2
        + [jax.ShapeDtypeStruct((1, NBP), jnp.int32)],
    )(logitsT)
    beid_1d = beid.reshape(NBP)

    nrow_m = N // BR
    rspec = pl.BlockSpec((BR, H), lambda e, r: (r, 0))
    base = pl.pallas_call(
        _shared_kernel,
        grid=(NS, nrow_m),
        in_specs=[rspec, rspec,
                  pl.BlockSpec((1, H, I), lambda e, r: (e, 0, 0)),
                  pl.BlockSpec((1, H, I), lambda e, r: (e, 0, 0)),
                  pl.BlockSpec((1, I, H), lambda e, r: (e, 0, 0))],
        out_specs=pl.BlockSpec((N, H), lambda e, r: (0, 0)),
        out_shape=jax.ShapeDtypeStruct((N, H), _f32),
    )(xn2, h, all_gate_t[NR:], all_up_t[NR:], all_down_t[NR:])

    i0x = _qrow_indices(pos0)
    i1x = _qrow_indices(pos1)
    xs2 = _sc_scatter_rows(xn2.reshape(N2, HQ), i0x, i1x)
    xs = xs2.reshape(PN, H)

    ys = pl.pallas_call(
        _grouped_ffn_kernel,
        grid_spec=pltpu.PrefetchScalarGridSpec(
            num_scalar_prefetch=1,
            grid=(NBP,),
            in_specs=[
                pl.BlockSpec((BKR, H), lambda i, s: (i, 0)),
                pl.BlockSpec((1, H, I),
                             lambda i, s: (jnp.maximum(s[i], 0), 0, 0)),
                pl.BlockSpec((1, H, I),
                             lambda i, s: (jnp.maximum(s[i], 0), 0, 0)),
                pl.BlockSpec((1, I, H),
                             lambda i, s: (jnp.maximum(s[i], 0), 0, 0)),
            ],
            out_specs=pl.BlockSpec((BKR, H), lambda i, s: (i, 0)),
        ),
        out_shape=jax.ShapeDtypeStruct((PN, H), _f32),
    )(beid_1d, xs, all_gate_t[:NR], all_up_t[:NR], all_down_t[:NR])

    ys2 = ys.reshape(PN2, HQ)
    g0 = _sc_gather_rows(ys2, i0x).reshape(N, H)
    g1 = _sc_gather_rows(ys2, i1x).reshape(N, H)
    wspec = pl.BlockSpec((1, BQ), lambda i: (0, i))
    out = pl.pallas_call(
        _combine_kernel,
        grid=(nrow,),
        in_specs=[row_spec, row_spec, row_spec, wspec, wspec],
        out_specs=row_spec,
        out_shape=jax.ShapeDtypeStruct((N, H), _f32),
        compiler_params=pltpu.CompilerParams(
            dimension_semantics=("parallel",)),
    )(base, g0, g1, w0, w1)

    return out.reshape(B, T, H)


# speedup vs baseline: 1.0507x; 1.0167x over previous
import jax
import jax.numpy as jnp
from jax.experimental import pallas as pl
from jax.experimental.pallas import tpu as pltpu
from jax.experimental.pallas import tpu_sc as plsc

B, T, H = 2, 2048, 1024
L = H // 4
I = int(H * 2.0)
NS = 2
NR = 8 - NS
NE = NR + NS
TOPK = 2
BASE = 10000.0
SCALE = 1.0
EPS = 1e-5
N = B * T

BQ = 256
BR = 256
RL = 128
BKR = 128
NBP = (N * TOPK) // BKR + NR
PN = NBP * BKR
GW = 128

_f32 = jnp.float32
_bf16 = jnp.bfloat16


def _ln(x, w):
    mu = jnp.mean(x, axis=1, keepdims=True)
    xc = x - mu
    var = jnp.mean(xc * xc, axis=1, keepdims=True)
    return xc * jax.lax.rsqrt(var + EPS) * w


def _prep_kernel(x_ref, ln1w_ref, wqd_ref, wrq_ref, wrk_ref, wkvd_ref,
                 wvu_ref, cos_ref, sin_ref, q_ref, k_ref, v_ref):
    x = x_ref[...]
    xb = _ln(x, ln1w_ref[...]).astype(_bf16)
    ql = jnp.dot(xb, wqd_ref[...], preferred_element_type=_f32).astype(_bf16)
    qr = jnp.dot(ql, wrq_ref[...], preferred_element_type=_f32)
    kr = jnp.dot(xb, wrk_ref[...], preferred_element_type=_f32)
    kv = jnp.dot(xb, wkvd_ref[...], preferred_element_type=_f32).astype(_bf16)
    v = jnp.dot(kv, wvu_ref[...], preferred_element_type=_f32)
    cos = cos_ref[...]
    sin = sin_ref[...]

    def rope(t):
        t1 = t[:, :H // 2]
        t2 = t[:, H // 2:]
        rot = jnp.concatenate([-t2, t1], axis=1)
        return t * cos + rot * sin

    q_ref[...] = rope(qr).astype(_bf16)
    k_ref[...] = rope(kr).astype(_bf16)
    v_ref[...] = v.astype(_bf16)


def _attn_kernel(q_ref, k_ref, v_ref, o_ref):
    qi = pl.program_id(1)
    q = q_ref[0]
    k = k_ref[0]
    s = jax.lax.dot_general(q, k, (((1,), (1,)), ((), ())),
                            preferred_element_type=_f32) * (1.0 / 32.0)
    row = qi * BQ + jax.lax.broadcasted_iota(jnp.int32, (BQ, T), 0)
    col = jax.lax.broadcasted_iota(jnp.int32, (BQ, T), 1)
    s = jnp.where(row >= col, s, -1e30)
    m = jnp.max(s, axis=1, keepdims=True)
    p = jnp.exp(s - m)
    p = p / jnp.sum(p, axis=1, keepdims=True)
    o_ref[0] = jnp.dot(p.astype(_bf16), v_ref[0],
                       preferred_element_type=_f32).astype(_bf16)


def _post_kernel(y_ref, x_ref, wo_ref, ln2w_ref, wrt_ref, bias_ref,
                 h_ref, xn2_ref, logits_ref, lt_ref):
    h = x_ref[...] + jnp.dot(y_ref[...], wo_ref[...],
                             preferred_element_type=_f32)
    h_ref[...] = h
    xn2 = _ln(h, ln2w_ref[...])
    xn2_ref[...] = xn2
    logits = jnp.dot(xn2, wrt_ref[...],
                     preferred_element_type=_f32) + bias_ref[...]
    logits_ref[...] = logits
    lt_ref[...] = jnp.transpose(logits[:, :8])


def _shared_kernel(xn2_ref, h_ref, gate_ref, up_ref, down_ref, out_ref):
    e = pl.program_id(0)
    r = pl.program_id(1)
    xb = xn2_ref[...].astype(_bf16)
    a = jnp.dot(xb, gate_ref[0], preferred_element_type=_f32)
    b = jnp.dot(xb, up_ref[0], preferred_element_type=_f32)
    h1 = (a * jax.nn.sigmoid(a) * b).astype(_bf16)
    contrib = jnp.dot(h1, down_ref[0],
                      preferred_element_type=_f32) * (1.0 / NS)
    rows = pl.ds(r * BR, BR)

    @pl.when(e == 0)
    def _init():
        out_ref[rows, :] = h_ref[...] + contrib

    @pl.when(e != 0)
    def _acc():
        out_ref[rows, :] += contrib


def _cumsum_lanes(x):
    n = x.shape[-1]
    sh = 1
    while sh < n:
        shifted = jnp.concatenate(
            [jnp.zeros((x.shape[0], sh), x.dtype), x[:, :n - sh]], axis=1)
        x = x + shifted
        sh *= 2
    return x


def _routing_kernel(lt_ref, pos0_ref, pos1_ref, w0_ref, w1_ref, beid_ref):
    P = jax.nn.sigmoid(lt_ref[...])
    eio = jax.lax.broadcasted_iota(jnp.int32, (8, N), 0)
    valid = eio < NR
    Pm = jnp.where(valid, P, -1.0)
    m1 = jnp.max(Pm, axis=0, keepdims=True)
    e0 = jnp.min(jnp.where(Pm == m1, eio, NR), axis=0, keepdims=True)
    mask0 = eio == e0
    Pm2 = jnp.where(mask0, -1.0, Pm)
    m2 = jnp.max(Pm2, axis=0, keepdims=True)
    e1 = jnp.min(jnp.where(Pm2 == m2, eio, NR), axis=0, keepdims=True)
    mask1 = eio == e1

    ind0 = mask0.astype(jnp.int32)
    ind1 = mask1.astype(jnp.int32)
    c0 = _cumsum_lanes(ind0) - ind0
    c1 = _cumsum_lanes(ind1) - ind1
    tot0 = jnp.sum(ind0, axis=1, keepdims=True)
    tot1 = jnp.sum(ind1, axis=1, keepdims=True)
    cnt = tot0 + tot1
    pcnt = ((cnt + BKR - 1) // BKR) * BKR

    offs = [jnp.zeros((1, 1), jnp.int32)]
    for e in range(1, NR):
        offs.append(offs[-1] + pcnt[e - 1:e, :])
    offs += [offs[-1] + pcnt[NR - 1:NR, :]] * (8 - NR)
    poff = jnp.concatenate(offs, axis=0)

    rank0 = jnp.sum(jnp.where(mask0, c0, 0), axis=0, keepdims=True)
    rank1 = jnp.sum(jnp.where(mask1, tot0 + c1, 0), axis=0, keepdims=True)
    base0 = jnp.sum(jnp.where(mask0, poff, 0), axis=0, keepdims=True)
    base1 = jnp.sum(jnp.where(mask1, poff, 0), axis=0, keepdims=True)
    pos0_ref[...] = base0 + rank0
    pos1_ref[...] = base1 + rank1
    w0_ref[...] = m1
    w1_ref[...] = m2

    bio = jax.lax.broadcasted_iota(jnp.int32, (1, NBP), 1)
    bstart = bio * BKR
    eid = jnp.full((1, NBP), -1, jnp.int32)
    for e in range(NR):
        pe = poff[e:e + 1, :]
        in_e = (bstart >= pe) & (bstart < pe + pcnt[e:e + 1, :])
        eid = jnp.where(in_e, e, eid)
    beid_ref[...] = eid


def _grouped_ffn_kernel(s_ref, xs_ref, gate_ref, up_ref, down_ref, ys_ref):
    eid = s_ref[pl.program_id(0)]

    @pl.when(eid >= 0)
    def _compute():
        xb = xs_ref[...].astype(_bf16)
        a = jnp.dot(xb, gate_ref[0], preferred_element_type=_f32)
        b = jnp.dot(xb, up_ref[0], preferred_element_type=_f32)
        h1 = (a * jax.nn.sigmoid(a) * b).astype(_bf16)
        ys_ref[...] = jnp.dot(h1, down_ref[0], preferred_element_type=_f32)

    @pl.when(eid < 0)
    def _skip():
        ys_ref[...] = jnp.zeros((BKR, H), _f32)


def _combine_kernel(base_ref, g0_ref, g1_ref, w0_ref, w1_ref, out_ref):
    w0 = jnp.transpose(w0_ref[...])
    w1 = jnp.transpose(w1_ref[...])
    out_ref[...] = (base_ref[...]
                    + w0 * g0_ref[...].astype(_f32)
                    + w1 * g1_ref[...].astype(_f32))


HQ = H // 4
N2 = 4 * N
PN2 = 4 * PN


def _sc_mesh():
    return plsc.VectorSubcoreMesh(core_axis_name="core",
                                  subcore_axis_name="subcore")


def _sc_scatter_rows(x2, i0x, i1x):
    @pl.kernel(out_type=jax.ShapeDtypeStruct((PN2, HQ), _f32),
               mesh=_sc_mesh())
    def _k(x_hbm, p0_hbm, p1_hbm, o_hbm):
        def body(x_vmem, i0_vmem, i1_vmem):
            pltpu.sync_copy(x_vmem, o_hbm.at[i0_vmem.at[0]])
            pltpu.sync_copy(x_vmem, o_hbm.at[i1_vmem.at[0]])

        pltpu.emit_pipeline(
            body,
            grid=(N2 // GW,),
            in_specs=[
                pl.BlockSpec((GW, HQ), lambda i: (i, 0)),
                pl.BlockSpec((1, GW), lambda i: (0, i)),
                pl.BlockSpec((1, GW), lambda i: (0, i)),
            ],
            out_specs=[],
            core_axis_name=("core", "subcore"),
            dimension_semantics=(pltpu.PARALLEL,),
        )(x_hbm, p0_hbm, p1_hbm)

    return _k(x2, i0x, i1x)


def _sc_gather_rows(y2, ix):
    @pl.kernel(out_type=jax.ShapeDtypeStruct((N2, HQ), _f32),
               mesh=_sc_mesh())
    def _k(y_hbm, p_hbm, o_hbm):
        def body(i_vmem, o_vmem):
            pltpu.sync_copy(y_hbm.at[i_vmem.at[0]], o_vmem)

        pltpu.emit_pipeline(
            body,
            grid=(N2 // GW,),
            in_specs=[pl.BlockSpec((1, GW), lambda i: (0, i))],
            out_specs=[pl.BlockSpec((GW, HQ), lambda i: (i, 0))],
            core_axis_name=("core", "subcore"),
            dimension_semantics=(pltpu.PARALLEL,),
        )(p_hbm, o_hbm)

    return _k(y2, ix)


def _qrow_indices(pos):
    p = pos.reshape(N, 1)
    return (4 * p + jnp.arange(4, dtype=jnp.int32)[None, :]).reshape(1, N2)


def kernel(x, ln1_w, ln2_w, W_kvd, W_qd, W_ku, W_qu, W_vu, W_rk, W_rq, W_o,
           sh_gate, sh_up, sh_down, rt_gate, rt_up, rt_down, W_router,
           routing_bias):
    del W_ku, W_qu
    xf = x.reshape(N, H)

    wqd_t = W_qd.T.astype(_bf16)
    wrq_t = W_rq.T.astype(_bf16)
    wrk_t = W_rk.T.astype(_bf16)
    wkvd_t = W_kvd.T.astype(_bf16)
    wvu_t = W_vu.T.astype(_bf16)
    wo_t = W_o.T.astype(_bf16)
    ln1w2 = ln1_w.reshape(1, H)
    ln2w2 = ln2_w.reshape(1, H)

    inv_freq = 1.0 / BASE ** (jnp.arange(0, H, 2, dtype=_f32) / H)
    tt = jnp.arange(T, dtype=_f32)
    freqs = tt[:, None] * inv_freq[None, :]
    emb = jnp.concatenate([freqs, freqs], axis=-1)
    cos = jnp.cos(emb) * SCALE
    sin = jnp.sin(emb) * SCALE

    wrt_t = jnp.zeros((H, RL), _f32).at[:, :NR].set(W_router.T)
    bias_p = jnp.full((1, RL), -1e30, _f32).at[0, :NR].set(routing_bias)

    all_gate_t = jnp.concatenate([rt_gate, sh_gate]).transpose(0, 2, 1).astype(_bf16)
    all_up_t = jnp.concatenate([rt_up, sh_up]).transpose(0, 2, 1).astype(_bf16)
    all_down_t = jnp.concatenate([rt_down, sh_down]).transpose(0, 2, 1).astype(_bf16)

    nrow = N // BQ
    row_spec = pl.BlockSpec((BQ, H), lambda i: (i, 0))
    full = lambda shape: pl.BlockSpec(shape, lambda i: tuple(0 for _ in shape))
    cs_spec = pl.BlockSpec((BQ, H), lambda i: (i % (T // BQ), 0))
    q, k, v = pl.pallas_call(
        _prep_kernel,
        grid=(nrow,),
        in_specs=[row_spec, full((1, H)), full((H, L)), full((L, H)),
                  full((H, H)), full((H, L)), full((L, H)), cs_spec, cs_spec],
        out_specs=[row_spec, row_spec, row_spec],
        out_shape=[jax.ShapeDtypeStruct((N, H), _bf16)] * 3,
        compiler_params=pltpu.CompilerParams(
            dimension_semantics=("parallel",)),
    )(xf, ln1w2, wqd_t, wrq_t, wrk_t, wkvd_t, wvu_t, cos, sin)

    q3 = q.reshape(B, T, H)
    k3 = k.reshape(B, T, H)
    v3 = v.reshape(B, T, H)
    qb_spec = pl.BlockSpec((1, BQ, H), lambda b, i: (b, i, 0))
    kv_spec = pl.BlockSpec((1, T, H), lambda b, i: (b, 0, 0))
    y = pl.pallas_call(
        _attn_kernel,
        grid=(B, T // BQ),
        in_specs=[qb_spec, kv_spec, kv_spec],
        out_specs=qb_spec,
        out_shape=jax.ShapeDtypeStruct((B, T, H), _bf16),
        compiler_params=pltpu.CompilerParams(
            dimension_semantics=("parallel", "parallel")),
    )(q3, k3, v3)

    h, xn2, logits, logitsT = pl.pallas_call(
        _post_kernel,
        grid=(nrow,),
        in_specs=[row_spec, row_spec, full((H, H)), full((1, H)),
                  full((H, RL)), full((1, RL))],
        out_specs=[row_spec, row_spec,
                   pl.BlockSpec((BQ, RL), lambda i: (i, 0)),
                   pl.BlockSpec((8, BQ), lambda i: (0, i))],
        out_shape=[jax.ShapeDtypeStruct((N, H), _f32),
                   jax.ShapeDtypeStruct((N, H), _f32),
                   jax.ShapeDtypeStruct((N, RL), _f32),
                   jax.ShapeDtypeStruct((8, N), _f32)],
        compiler_params=pltpu.CompilerParams(
            dimension_semantics=("parallel",)),
    )(y.reshape(N, H), xf, wo_t, ln2w2, wrt_t, bias_p)

    pos0, pos1, w0, w1, beid = pl.pallas_call(
        _routing_kernel,
        grid=(1,),
        in_specs=[pl.BlockSpec((8, N), lambda i: (0, 0))],
        out_specs=[pl.BlockSpec((1, N), lambda i: (0, 0))] * 4
        + [pl.BlockSpec((1, NBP), lambda i: (0, 0))],
        out_shape=[jax.ShapeDtypeStruct((1, N), jnp.int32)] * 2
        + [jax.ShapeDtypeStruct((1, N), _f32)] * 2
        + [jax.ShapeDtypeStruct((1, NBP), jnp.int32)],
    )(logitsT)
    beid_1d = beid.reshape(NBP)

    nrow_m = N // BR
    rspec = pl.BlockSpec((BR, H), lambda e, r: (r, 0))
    base = pl.pallas_call(
        _shared_kernel,
        grid=(NS, nrow_m),
        in_specs=[rspec, rspec,
                  pl.BlockSpec((1, H, I), lambda e, r: (e, 0, 0)),
                  pl.BlockSpec((1, H, I), lambda e, r: (e, 0, 0)),
                  pl.BlockSpec((1, I, H), lambda e, r: (e, 0, 0))],
        out_specs=pl.BlockSpec((N, H), lambda e, r: (0, 0)),
        out_shape=jax.ShapeDtypeStruct((N, H), _f32),
    )(xn2, h, all_gate_t[NR:], all_up_t[NR:], all_down_t[NR:])

    i0x = _qrow_indices(pos0)
    i1x = _qrow_indices(pos1)
    xs2 = _sc_scatter_rows(xn2.reshape(N2, HQ), i0x, i1x)
    xs = xs2.reshape(PN, H)

    ys = pl.pallas_call(
        _grouped_ffn_kernel,
        grid_spec=pltpu.PrefetchScalarGridSpec(
            num_scalar_prefetch=1,
            grid=(NBP,),
            in_specs=[
                pl.BlockSpec((BKR, H), lambda i, s: (i, 0)),
                pl.BlockSpec((1, H, I),
                             lambda i, s: (jnp.maximum(s[i], 0), 0, 0)),
                pl.BlockSpec((1, H, I),
                             lambda i, s: (jnp.maximum(s[i], 0), 0, 0)),
                pl.BlockSpec((1, I, H),
                             lambda i, s: (jnp.maximum(s[i], 0), 0, 0)),
            ],
            out_specs=pl.BlockSpec((BKR, H), lambda i, s: (i, 0)),
        ),
        out_shape=jax.ShapeDtypeStruct((PN, H), _f32),
    )(beid_1d, xs, all_gate_t[:NR], all_up_t[:NR], all_down_t[:NR])

    ys2 = ys.reshape(PN2, HQ)
    g0 = _sc_gather_rows(ys2, i0x).reshape(N, H)
    g1 = _sc_gather_rows(ys2, i1x).reshape(N, H)
    wspec = pl.BlockSpec((1, BQ), lambda i: (0, i))
    out = pl.pallas_call(
        _combine_kernel,
        grid=(nrow,),
        in_specs=[row_spec, row_spec, row_spec, wspec, wspec],
        out_specs=row_spec,
        out_shape=jax.ShapeDtypeStruct((N, H), _f32),
        compiler_params=pltpu.CompilerParams(
            dimension_semantics=("parallel",)),
    )(base, g0, g1, w0, w1)

    return out.reshape(B, T, H)


# speedup vs baseline: 1.0564x; 1.0055x over previous
import jax
import jax.numpy as jnp
from jax.experimental import pallas as pl
from jax.experimental.pallas import tpu as pltpu
from jax.experimental.pallas import tpu_sc as plsc

B, T, H = 2, 2048, 1024
L = H // 4
I = int(H * 2.0)
NS = 2
NR = 8 - NS
NE = NR + NS
TOPK = 2
BASE = 10000.0
SCALE = 1.0
EPS = 1e-5
N = B * T

BQ = 256
BR = 256
RL = 128
BKR = 128
NBP = (N * TOPK) // BKR + NR
PN = NBP * BKR
GW = 128

_f32 = jnp.float32
_bf16 = jnp.bfloat16


def _ln(x, w):
    mu = jnp.mean(x, axis=1, keepdims=True)
    xc = x - mu
    var = jnp.mean(xc * xc, axis=1, keepdims=True)
    return xc * jax.lax.rsqrt(var + EPS) * w


def _prep_kernel(x_ref, ln1w_ref, wqd_ref, wrq_ref, wrk_ref, wkvd_ref,
                 wvu_ref, cos_ref, sin_ref, q_ref, k_ref, v_ref):
    x = x_ref[...]
    xb = _ln(x, ln1w_ref[...]).astype(_bf16)
    ql = jnp.dot(xb, wqd_ref[...], preferred_element_type=_f32).astype(_bf16)
    qr = jnp.dot(ql, wrq_ref[...], preferred_element_type=_f32)
    kr = jnp.dot(xb, wrk_ref[...], preferred_element_type=_f32)
    kv = jnp.dot(xb, wkvd_ref[...], preferred_element_type=_f32).astype(_bf16)
    v = jnp.dot(kv, wvu_ref[...], preferred_element_type=_f32)
    cos = cos_ref[...]
    sin = sin_ref[...]

    def rope(t):
        t1 = t[:, :H // 2]
        t2 = t[:, H // 2:]
        rot = jnp.concatenate([-t2, t1], axis=1)
        return t * cos + rot * sin

    q_ref[...] = rope(qr).astype(_bf16)
    k_ref[...] = rope(kr).astype(_bf16)
    v_ref[...] = v.astype(_bf16)


def _attn_kernel(q_ref, k_ref, v_ref, o_ref):
    qi = pl.program_id(1)
    q = q_ref[0]
    k = k_ref[0]
    s = jax.lax.dot_general(q, k, (((1,), (1,)), ((), ())),
                            preferred_element_type=_f32) * (1.0 / 32.0)
    row = qi * BQ + jax.lax.broadcasted_iota(jnp.int32, (BQ, T), 0)
    col = jax.lax.broadcasted_iota(jnp.int32, (BQ, T), 1)
    s = jnp.where(row >= col, s, -1e30)
    m = jnp.max(s, axis=1, keepdims=True)
    p = jnp.exp(s - m)
    p = p / jnp.sum(p, axis=1, keepdims=True)
    o_ref[0] = jnp.dot(p.astype(_bf16), v_ref[0],
                       preferred_element_type=_f32).astype(_bf16)


def _post_kernel(y_ref, x_ref, wo_ref, ln2w_ref, wrt_ref, bias_ref,
                 h_ref, xn2_ref, logits_ref, lt_ref):
    h = x_ref[...] + jnp.dot(y_ref[...], wo_ref[...],
                             preferred_element_type=_f32)
    h_ref[...] = h
    xn2 = _ln(h, ln2w_ref[...])
    xn2_ref[...] = xn2
    logits = jnp.dot(xn2, wrt_ref[...],
                     preferred_element_type=_f32) + bias_ref[...]
    logits_ref[...] = logits
    lt_ref[...] = jnp.transpose(logits[:, :8])


def _shared_single_kernel(xn2_ref, base_ref, gate_ref, up_ref, down_ref,
                          out_ref):
    xb = xn2_ref[...].astype(_bf16)
    a = jnp.dot(xb, gate_ref[...], preferred_element_type=_f32)
    b = jnp.dot(xb, up_ref[...], preferred_element_type=_f32)
    h1 = (a * jax.nn.sigmoid(a) * b).astype(_bf16)
    out_ref[...] = base_ref[...] + jnp.dot(
        h1, down_ref[...], preferred_element_type=_f32) * (1.0 / NS)


def _cumsum_lanes(x):
    n = x.shape[-1]
    sh = 1
    while sh < n:
        shifted = jnp.concatenate(
            [jnp.zeros((x.shape[0], sh), x.dtype), x[:, :n - sh]], axis=1)
        x = x + shifted
        sh *= 2
    return x


def _routing_kernel(lt_ref, pos0_ref, pos1_ref, w0_ref, w1_ref, beid_ref):
    P = jax.nn.sigmoid(lt_ref[...])
    eio = jax.lax.broadcasted_iota(jnp.int32, (8, N), 0)
    valid = eio < NR
    Pm = jnp.where(valid, P, -1.0)
    m1 = jnp.max(Pm, axis=0, keepdims=True)
    e0 = jnp.min(jnp.where(Pm == m1, eio, NR), axis=0, keepdims=True)
    mask0 = eio == e0
    Pm2 = jnp.where(mask0, -1.0, Pm)
    m2 = jnp.max(Pm2, axis=0, keepdims=True)
    e1 = jnp.min(jnp.where(Pm2 == m2, eio, NR), axis=0, keepdims=True)
    mask1 = eio == e1

    ind0 = mask0.astype(jnp.int32)
    ind1 = mask1.astype(jnp.int32)
    c0 = _cumsum_lanes(ind0) - ind0
    c1 = _cumsum_lanes(ind1) - ind1
    tot0 = jnp.sum(ind0, axis=1, keepdims=True)
    tot1 = jnp.sum(ind1, axis=1, keepdims=True)
    cnt = tot0 + tot1
    pcnt = ((cnt + BKR - 1) // BKR) * BKR

    offs = [jnp.zeros((1, 1), jnp.int32)]
    for e in range(1, NR):
        offs.append(offs[-1] + pcnt[e - 1:e, :])
    offs += [offs[-1] + pcnt[NR - 1:NR, :]] * (8 - NR)
    poff = jnp.concatenate(offs, axis=0)

    rank0 = jnp.sum(jnp.where(mask0, c0, 0), axis=0, keepdims=True)
    rank1 = jnp.sum(jnp.where(mask1, tot0 + c1, 0), axis=0, keepdims=True)
    base0 = jnp.sum(jnp.where(mask0, poff, 0), axis=0, keepdims=True)
    base1 = jnp.sum(jnp.where(mask1, poff, 0), axis=0, keepdims=True)
    pos0_ref[...] = base0 + rank0
    pos1_ref[...] = base1 + rank1
    w0_ref[...] = m1
    w1_ref[...] = m2

    bio = jax.lax.broadcasted_iota(jnp.int32, (1, NBP), 1)
    bstart = bio * BKR
    eid = jnp.full((1, NBP), -1, jnp.int32)
    for e in range(NR):
        pe = poff[e:e + 1, :]
        in_e = (bstart >= pe) & (bstart < pe + pcnt[e:e + 1, :])
        eid = jnp.where(in_e, e, eid)
    beid_ref[...] = eid


def _grouped_ffn_kernel(s_ref, xs_ref, gate_ref, up_ref, down_ref, ys_ref):
    eid = s_ref[pl.program_id(0)]

    @pl.when(eid >= 0)
    def _compute():
        xb = xs_ref[...].astype(_bf16)
        a = jnp.dot(xb, gate_ref[0], preferred_element_type=_f32)
        b = jnp.dot(xb, up_ref[0], preferred_element_type=_f32)
        h1 = (a * jax.nn.sigmoid(a) * b).astype(_bf16)
        ys_ref[...] = jnp.dot(h1, down_ref[0], preferred_element_type=_f32)

    @pl.when(eid < 0)
    def _skip():
        ys_ref[...] = jnp.zeros((BKR, H), _f32)


def _combine_kernel(base_ref, g0_ref, g1_ref, w0_ref, w1_ref, out_ref):
    w0 = jnp.transpose(w0_ref[...])
    w1 = jnp.transpose(w1_ref[...])
    out_ref[...] = (base_ref[...]
                    + w0 * g0_ref[...]
                    + w1 * g1_ref[...])


HQ = H // 4
N2 = 4 * N
PN2 = 4 * PN


def _sc_mesh():
    return plsc.VectorSubcoreMesh(core_axis_name="core",
                                  subcore_axis_name="subcore")


def _sc_scatter_rows(x2, i0x, i1x):
    @pl.kernel(out_type=jax.ShapeDtypeStruct((PN2, HQ), _f32),
               mesh=_sc_mesh())
    def _k(x_hbm, p0_hbm, p1_hbm, o_hbm):
        def body(x_vmem, i0_vmem, i1_vmem):
            pltpu.sync_copy(x_vmem, o_hbm.at[i0_vmem.at[0]])
            pltpu.sync_copy(x_vmem, o_hbm.at[i1_vmem.at[0]])

        pltpu.emit_pipeline(
            body,
            grid=(N2 // GW,),
            in_specs=[
                pl.BlockSpec((GW, HQ), lambda i: (i, 0)),
                pl.BlockSpec((1, GW), lambda i: (0, i)),
                pl.BlockSpec((1, GW), lambda i: (0, i)),
            ],
            out_specs=[],
            core_axis_name=("core", "subcore"),
            dimension_semantics=(pltpu.PARALLEL,),
        )(x_hbm, p0_hbm, p1_hbm)

    return _k(x2, i0x, i1x)


def _sc_gather_rows(y2, ix, nrows):
    @pl.kernel(out_type=jax.ShapeDtypeStruct((nrows, HQ), _f32),
               mesh=_sc_mesh())
    def _k(y_hbm, p_hbm, o_hbm):
        def body(i_vmem, o_vmem):
            pltpu.sync_copy(y_hbm.at[i_vmem.at[0]], o_vmem)

        pltpu.emit_pipeline(
            body,
            grid=(nrows // GW,),
            in_specs=[pl.BlockSpec((1, GW), lambda i: (0, i))],
            out_specs=[pl.BlockSpec((GW, HQ), lambda i: (i, 0))],
            core_axis_name=("core", "subcore"),
            dimension_semantics=(pltpu.PARALLEL,),
        )(p_hbm, o_hbm)

    return _k(y2, ix)


def _subrow_indices(pos, k):
    p = pos.reshape(N, 1)
    return (k * p + jnp.arange(k, dtype=jnp.int32)[None, :]).reshape(1, k * N)


def kernel(x, ln1_w, ln2_w, W_kvd, W_qd, W_ku, W_qu, W_vu, W_rk, W_rq, W_o,
           sh_gate, sh_up, sh_down, rt_gate, rt_up, rt_down, W_router,
           routing_bias):
    del W_ku, W_qu
    xf = x.reshape(N, H)

    wqd_t = W_qd.T.astype(_bf16)
    wrq_t = W_rq.T.astype(_bf16)
    wrk_t = W_rk.T.astype(_bf16)
    wkvd_t = W_kvd.T.astype(_bf16)
    wvu_t = W_vu.T.astype(_bf16)
    wo_t = W_o.T.astype(_bf16)
    ln1w2 = ln1_w.reshape(1, H)
    ln2w2 = ln2_w.reshape(1, H)

    inv_freq = 1.0 / BASE ** (jnp.arange(0, H, 2, dtype=_f32) / H)
    tt = jnp.arange(T, dtype=_f32)
    freqs = tt[:, None] * inv_freq[None, :]
    emb = jnp.concatenate([freqs, freqs], axis=-1)
    cos = jnp.cos(emb) * SCALE
    sin = jnp.sin(emb) * SCALE

    wrt_t = jnp.zeros((H, RL), _f32).at[:, :NR].set(W_router.T)
    bias_p = jnp.full((1, RL), -1e30, _f32).at[0, :NR].set(routing_bias)

    all_gate_t = jnp.concatenate([rt_gate, sh_gate]).transpose(0, 2, 1).astype(_bf16)
    all_up_t = jnp.concatenate([rt_up, sh_up]).transpose(0, 2, 1).astype(_bf16)
    all_down_t = jnp.concatenate([rt_down, sh_down]).transpose(0, 2, 1).astype(_bf16)

    nrow = N // BQ
    row_spec = pl.BlockSpec((BQ, H), lambda i: (i, 0))
    full = lambda shape: pl.BlockSpec(shape, lambda i: tuple(0 for _ in shape))
    cs_spec = pl.BlockSpec((BQ, H), lambda i: (i % (T // BQ), 0))
    q, k, v = pl.pallas_call(
        _prep_kernel,
        grid=(nrow,),
        in_specs=[row_spec, full((1, H)), full((H, L)), full((L, H)),
                  full((H, H)), full((H, L)), full((L, H)), cs_spec, cs_spec],
        out_specs=[row_spec, row_spec, row_spec],
        out_shape=[jax.ShapeDtypeStruct((N, H), _bf16)] * 3,
        compiler_params=pltpu.CompilerParams(
            dimension_semantics=("parallel",)),
    )(xf, ln1w2, wqd_t, wrq_t, wrk_t, wkvd_t, wvu_t, cos, sin)

    q3 = q.reshape(B, T, H)
    k3 = k.reshape(B, T, H)
    v3 = v.reshape(B, T, H)
    qb_spec = pl.BlockSpec((1, BQ, H), lambda b, i: (b, i, 0))
    kv_spec = pl.BlockSpec((1, T, H), lambda b, i: (b, 0, 0))
    y = pl.pallas_call(
        _attn_kernel,
        grid=(B, T // BQ),
        in_specs=[qb_spec, kv_spec, kv_spec],
        out_specs=qb_spec,
        out_shape=jax.ShapeDtypeStruct((B, T, H), _bf16),
        compiler_params=pltpu.CompilerParams(
            dimension_semantics=("parallel", "parallel")),
    )(q3, k3, v3)

    h, xn2, logits, logitsT = pl.pallas_call(
        _post_kernel,
        grid=(nrow,),
        in_specs=[row_spec, row_spec, full((H, H)), full((1, H)),
                  full((H, RL)), full((1, RL))],
        out_specs=[row_spec, row_spec,
                   pl.BlockSpec((BQ, RL), lambda i: (i, 0)),
                   pl.BlockSpec((8, BQ), lambda i: (0, i))],
        out_shape=[jax.ShapeDtypeStruct((N, H), _f32),
                   jax.ShapeDtypeStruct((N, H), _f32),
                   jax.ShapeDtypeStruct((N, RL), _f32),
                   jax.ShapeDtypeStruct((8, N), _f32)],
        compiler_params=pltpu.CompilerParams(
            dimension_semantics=("parallel",)),
    )(y.reshape(N, H), xf, wo_t, ln2w2, wrt_t, bias_p)

    pos0, pos1, w0, w1, beid = pl.pallas_call(
        _routing_kernel,
        grid=(1,),
        in_specs=[pl.BlockSpec((8, N), lambda i: (0, 0))],
        out_specs=[pl.BlockSpec((1, N), lambda i: (0, 0))] * 4
        + [pl.BlockSpec((1, NBP), lambda i: (0, 0))],
        out_shape=[jax.ShapeDtypeStruct((1, N), jnp.int32)] * 2
        + [jax.ShapeDtypeStruct((1, N), _f32)] * 2
        + [jax.ShapeDtypeStruct((1, NBP), jnp.int32)],
    )(logitsT)
    beid_1d = beid.reshape(NBP)

    nrow_m = N // BR
    rspec = pl.BlockSpec((BR, H), lambda r: (r, 0))

    def _shared_call(base_in, j):
        return pl.pallas_call(
            _shared_single_kernel,
            grid=(nrow_m,),
            in_specs=[rspec, rspec, full((H, I)), full((H, I)),
                      full((I, H))],
            out_specs=rspec,
            out_shape=jax.ShapeDtypeStruct((N, H), _f32),
            compiler_params=pltpu.CompilerParams(
                dimension_semantics=("parallel",)),
        )(xn2, base_in, all_gate_t[NR + j], all_up_t[NR + j],
          all_down_t[NR + j])

    base1 = _shared_call(h, 0)

    i0x = _subrow_indices(pos0, 4)
    i1x = _subrow_indices(pos1, 4)
    xs2 = _sc_scatter_rows(xn2.reshape(N2, HQ), i0x, i1x)
    xs = xs2.reshape(PN, H)

    ys = pl.pallas_call(
        _grouped_ffn_kernel,
        grid_spec=pltpu.PrefetchScalarGridSpec(
            num_scalar_prefetch=1,
            grid=(NBP,),
            in_specs=[
                pl.BlockSpec((BKR, H), lambda i, s: (i, 0)),
                pl.BlockSpec((1, H, I),
                             lambda i, s: (jnp.maximum(s[i], 0), 0, 0)),
                pl.BlockSpec((1, H, I),
                             lambda i, s: (jnp.maximum(s[i], 0), 0, 0)),
                pl.BlockSpec((1, I, H),
                             lambda i, s: (jnp.maximum(s[i], 0), 0, 0)),
            ],
            out_specs=pl.BlockSpec((BKR, H), lambda i, s: (i, 0)),
        ),
        out_shape=jax.ShapeDtypeStruct((PN, H), _f32),
    )(beid_1d, xs, all_gate_t[:NR], all_up_t[:NR], all_down_t[:NR])

    base2 = _shared_call(base1, 1)
    ys2 = ys.reshape(4 * PN, HQ)
    g0 = _sc_gather_rows(ys2, i0x, N2).reshape(N, H)
    g1 = _sc_gather_rows(ys2, i1x, N2).reshape(N, H)
    wspec = pl.BlockSpec((1, BQ), lambda i: (0, i))
    gspec = pl.BlockSpec((BQ, H), lambda i: (i, 0))
    out = pl.pallas_call(
        _combine_kernel,
        grid=(nrow,),
        in_specs=[row_spec, gspec, gspec, wspec, wspec],
        out_specs=row_spec,
        out_shape=jax.ShapeDtypeStruct((N, H), _f32),
        compiler_params=pltpu.CompilerParams(
            dimension_semantics=("parallel",)),
    )(base2, g0, g1, w0, w1)

    return out.reshape(B, T, H)
